# baseline (speedup 1.0000x reference)
import functools

import jax
import jax.numpy as jnp
from jax import lax
from jax.experimental import pallas as pl
from jax.experimental.pallas import tpu as pltpu
from jax.experimental.pallas import tpu_sc as plsc

NC = 2
NSC = 16
NT = NC * NSC
CH = 128


def _ceil(a, b):
    return -(-a // b)


def _pad_block(n, pad, dtype):
    if isinstance(pad, tuple):
        return (jnp.arange(n, dtype=jnp.int32) % pad[1]).astype(dtype)
    return jnp.full((n,), pad, dtype)


def _pack_flat(arr, n_tiles, pad):
    e = arr.shape[0]
    per = _ceil(_ceil(e, n_tiles), CH) * CH
    total = n_tiles * per
    arr = jnp.concatenate([arr, _pad_block(total - e, pad, arr.dtype)])
    return arr.reshape(n_tiles, per // CH, CH), per // CH


def _pack_grouped(arr, n_groups, pad):
    eg = arr.shape[0] // n_groups
    per = _ceil(_ceil(eg, NSC), CH) * CH
    a = arr.reshape(n_groups, NSC, eg // NSC)
    padb = jnp.broadcast_to(
        _pad_block(per - eg // NSC, pad, arr.dtype),
        (n_groups, NSC, per - eg // NSC))
    a = jnp.concatenate([a, padb], axis=2)
    return a.reshape(n_groups, NSC, per // CH, CH), per // CH


def _sc_mesh():
    return plsc.VectorSubcoreMesh(core_axis_name="c", subcore_axis_name="s",
                                  num_cores=NC, num_subcores=NSC)


def _zero_fill_2d(zrow, acc, row0, n_blocks, f):
    def zbody(i, carry):
        for j in range(f // 16):
            zrow[i, pl.ds(16 * j, 16)] = jnp.zeros((16,), jnp.float32)
        return carry

    lax.fori_loop(0, 16, zbody, 0)

    def fbody(i, carry):
        pltpu.sync_copy(zrow, acc.at[pl.ds(row0 + 16 * i, 16)])
        return carry

    lax.fori_loop(0, n_blocks, fbody, 0)


def _agg_rows_full(table, src_pk, dst_pk, r_pad, f, n_chunks):
    rows_per = r_pad // NSC

    @functools.partial(
        pl.kernel,
        out_type=jax.ShapeDtypeStruct((2 * r_pad, f), jnp.float32),
        mesh=_sc_mesh(),
        scratch_types=[
            pltpu.VMEM((n_chunks, CH), jnp.int32),
            pltpu.VMEM((n_chunks, CH), jnp.int32),
            pltpu.VMEM((CH, f), jnp.float32),
            pltpu.VMEM((16, f), jnp.float32),
            pltpu.VMEM((16, f), jnp.float32),
            pltpu.VMEM_SHARED((r_pad, f), jnp.float32),
            pltpu.SemaphoreType.DMA,
        ],
    )
    def k(table_h, src_h, dst_h, out_h, sidx, didx, buf, zrow, tbuf, acc,
          sem):
        cid = lax.axis_index("c")
        sid = lax.axis_index("s")
        wid = cid * NSC + sid
        _zero_fill_2d(zrow, acc, sid * rows_per, rows_per // 16, f)
        pltpu.sync_copy(src_h.at[wid], sidx)
        pltpu.sync_copy(dst_h.at[wid], didx)
        plsc.subcore_barrier()

        def body(c, carry):
            pltpu.async_copy(table_h.at[sidx.at[c]], buf, sem).wait()
            pltpu.sync_copy(buf, acc.at[didx.at[c]], add=True)
            return carry

        lax.fori_loop(0, n_chunks, body, 0)
        plsc.subcore_barrier()

        def obody(i, carry):
            pltpu.sync_copy(
                acc.at[pl.ds(sid * rows_per + 16 * i, 16)], tbuf)
            pltpu.sync_copy(
                tbuf,
                out_h.at[pl.ds(cid * r_pad + sid * rows_per + 16 * i, 16)])
            return carry

        lax.fori_loop(0, rows_per // 16, obody, 0)

    return k(table, src_pk, dst_pk)


def _agg_rows_grouped(table, src_pk, dst_pk, ns_out, spr, nr, f, n_chunks):
    r_acc = spr * ns_out
    r_pad = _ceil(r_acc + 1, 256) * 256
    rows_per = r_pad // NSC
    out_per = r_acc // NSC
    src_pk = src_pk.reshape(2 * nr * NSC, n_chunks, CH)
    dst_pk = dst_pk.reshape(2 * nr * NSC, n_chunks, CH)

    @functools.partial(
        pl.kernel,
        out_type=jax.ShapeDtypeStruct((2 * nr * r_pad, f), jnp.float32),
        mesh=_sc_mesh(),
        scratch_types=[
            pltpu.VMEM((n_chunks, CH), jnp.int32),
            pltpu.VMEM((n_chunks, CH), jnp.int32),
            pltpu.VMEM((CH, f), jnp.float32),
            pltpu.VMEM((16, f), jnp.float32),
            pltpu.VMEM((16, f), jnp.float32),
            pltpu.VMEM_SHARED((r_pad, f), jnp.float32),
            pltpu.SemaphoreType.DMA,
        ],
    )
    def k(table_h, src_h, dst_h, out_h, sidx, didx, buf, zrow, tbuf, acc,
          sem):
        cid = lax.axis_index("c")
        sid = lax.axis_index("s")
        for r in range(nr):
            grp = cid * nr + r
            _zero_fill_2d(zrow, acc, sid * rows_per, rows_per // 16, f)
            pltpu.sync_copy(src_h.at[grp * NSC + sid], sidx)
            pltpu.sync_copy(dst_h.at[grp * NSC + sid], didx)
            plsc.subcore_barrier()

            def body(c, carry):
                pltpu.async_copy(table_h.at[sidx.at[c]], buf, sem).wait()
                pltpu.sync_copy(buf, acc.at[didx.at[c]], add=True)
                return carry

            lax.fori_loop(0, n_chunks, body, 0)
            plsc.subcore_barrier()

            def obody(i, carry):
                pltpu.sync_copy(
                    acc.at[pl.ds(sid * rows_per + 16 * i, 16)], tbuf)
                pltpu.sync_copy(
                    tbuf,
                    out_h.at[pl.ds(grp * r_pad + sid * rows_per + 16 * i,
                                   16)])
                return carry

            lax.fori_loop(0, rows_per // 16, obody, 0)
            plsc.subcore_barrier()

    out = k(table, src_pk, dst_pk)
    return out.reshape(2 * nr, r_pad, f)[:, :r_acc, :].reshape(
        2 * nr * r_acc, f)


def _agg_scalars(vals_pk, dst_pk, r_pad, n_chunks):
    rows_per = r_pad // NSC
    assert rows_per % 2048 == 0

    @functools.partial(
        pl.kernel,
        out_type=jax.ShapeDtypeStruct((2 * r_pad,), jnp.float32),
        mesh=_sc_mesh(),
        scratch_types=[
            pltpu.VMEM((n_chunks, CH), jnp.float32),
            pltpu.VMEM((n_chunks, CH), jnp.int32),
            pltpu.VMEM((2048,), jnp.float32),
            pltpu.VMEM((2048,), jnp.float32),
            pltpu.VMEM_SHARED((r_pad,), jnp.float32),
            pltpu.SemaphoreType.DMA,
        ],
    )
    def k(vals_h, dst_h, out_h, vbuf, didx, zbuf, obuf, acc, sem):
        cid = lax.axis_index("c")
        sid = lax.axis_index("s")
        wid = cid * NSC + sid

        def zbody(i, carry):
            zbuf[pl.ds(16 * i, 16)] = jnp.zeros((16,), jnp.float32)
            return carry

        lax.fori_loop(0, 128, zbody, 0)

        def fbody(i, carry):
            pltpu.sync_copy(
                zbuf, acc.at[pl.ds(sid * rows_per + 2048 * i, 2048)])
            return carry

        lax.fori_loop(0, rows_per // 2048, fbody, 0)
        pltpu.sync_copy(vals_h.at[wid], vbuf)
        pltpu.sync_copy(dst_h.at[wid], didx)
        plsc.subcore_barrier()

        def body(c, carry):
            pltpu.sync_copy(vbuf.at[c], acc.at[didx.at[c]], add=True)
            return carry

        lax.fori_loop(0, n_chunks, body, 0)
        plsc.subcore_barrier()

        def obody(i, carry):
            pltpu.sync_copy(
                acc.at[pl.ds(sid * rows_per + 2048 * i, 2048)], obuf)
            pltpu.sync_copy(
                obuf,
                out_h.at[pl.ds(cid * r_pad + sid * rows_per + 2048 * i,
                               2048)])
            return carry

        lax.fori_loop(0, rows_per // 2048, obody, 0)

    return k(vals_pk, dst_pk)


def _mm_kernel(x_ref, w_ref, o_ref):
    o_ref[...] = jnp.dot(x_ref[...], w_ref[...],
                         preferred_element_type=jnp.float32)


def _mm(x, w, block_m):
    m, kdim = x.shape
    _, n = w.shape
    assert m % block_m == 0
    return pl.pallas_call(
        _mm_kernel,
        out_shape=jax.ShapeDtypeStruct((m, n), jnp.float32),
        grid=(m // block_m,),
        in_specs=[
            pl.BlockSpec((block_m, kdim), lambda i: (i, 0)),
            pl.BlockSpec((kdim, n), lambda i: (0, 0)),
        ],
        out_specs=pl.BlockSpec((block_m, n), lambda i: (i, 0)),
    )(x, w)


def _mhsa(x, Wqkv, bqkv, Wout, bout, H):
    S, E = x.shape
    hd = E // H
    qkv = (x @ Wqkv.T + bqkv).reshape(S, 1, 3, H, hd).transpose(2, 0, 3, 1, 4)
    q, k, v = qkv[0], qkv[1], qkv[2]
    attn = jax.nn.softmax((q @ jnp.swapaxes(k, -2, -1)) / (hd ** 0.5), axis=-1)
    out = (attn @ v).transpose(0, 2, 1, 3).reshape(S, 1, E)
    return out @ Wout.T + bout


def kernel(x, edge_index, sub_x, sub_edge_index, orig_idx, W1, b1, W2, b2,
           Wp, bp, Ws, bs, Wqkv, bqkv, Wout, bout, Wf, bf):
    N = x.shape[0]
    S, NS, D = sub_x.shape
    ES = sub_edge_index.shape[2]
    NH = W1.shape[1]
    C = Wf.shape[1]
    KK = NS // 2
    f32 = jnp.float32

    src_s = sub_edge_index[:, 0, :]
    dst_s = sub_edge_index[:, 1, :]
    gids = jnp.arange(S, dtype=jnp.int32)[:, None]

    R1, R2 = S * NS, N
    RD = R1 + R2
    RD_pad = _ceil(RD + 1, NSC * 2048) * NSC * 2048
    dummy_d = RD
    dsub_flat = (gids * NS + dst_s).reshape(-1)
    deg_dst = jnp.concatenate([dsub_flat, R1 + edge_index[1]])
    deg_dst_pk, nch_d = _pack_flat(deg_dst, NT, ("spread", RD))
    ones_pk, _ = _pack_flat(jnp.ones(deg_dst.shape, f32), NT, 0.0)
    deg2 = _agg_scalars(ones_pk, deg_dst_pk, RD_pad, nch_d)
    deg_all = deg2[:RD] + deg2[RD_pad:RD_pad + RD] + 1.0
    dinv_sub = (deg_all[:R1] ** -0.5).reshape(S, NS)
    dinv_f = deg_all[R1:] ** -0.5

    dv = dinv_sub.reshape(-1, 1)
    SPR = 4
    NR = S // (2 * SPR)
    zrow_s = S * NS
    src_flat = (gids * NS + src_s).reshape(-1)
    dstm_flat = ((gids % SPR) * NS + dst_s).reshape(-1)
    src_pk, nch_s = _pack_grouped(src_flat, 2 * NR, zrow_s)
    dst_pk, _ = _pack_grouped(dstm_flat, 2 * NR, ("spread", SPR * NS))
    zpad8 = jnp.zeros((8, NH), f32)
    z0 = sub_x.reshape(S * NS, D) * dv
    y0 = _agg_rows_grouped(jnp.concatenate([z0, zpad8]), src_pk, dst_pk,
                           NS, SPR, NR, D, nch_s)
    x1 = jax.nn.relu(dv * _mm(y0 + z0, W1, 2000) + b1)

    Wcat = jnp.concatenate(
        [W2, Wp, jnp.zeros((NH, 7), f32)], axis=1)
    z1 = x1 * dv
    y1 = _agg_rows_grouped(jnp.concatenate([z1, zpad8]), src_pk, dst_pk,
                           NS, SPR, NR, NH, nch_s)
    h2 = dv * _mm(y1 + z1, Wcat, 2000)
    xn = jax.nn.relu(h2[:, :NH] + b2).reshape(S, NS, NH)
    score = (h2[:, NH] + bp[0]).reshape(S, NS)

    topv, perm = jax.lax.top_k(score, KK)
    x1r = x1.reshape(S, NS, NH)
    x_pool = jnp.take_along_axis(x1r, perm[:, :, None], axis=1) \
        * jnp.tanh(topv)[:, :, None]
    new_idx = jnp.full((S, NS), -1, jnp.int32).at[
        jnp.arange(S)[:, None], perm].set(
            jnp.broadcast_to(jnp.arange(KK, dtype=jnp.int32), (S, KK)))
    ns = jnp.take_along_axis(new_idx, src_s, axis=1)
    nd = jnp.take_along_axis(new_idx, dst_s, axis=1)
    emask = (ns >= 0) & (nd >= 0)
    ns = jnp.where(emask, ns, 0)
    nd = jnp.where(emask, nd, 0)

    emb1 = jnp.concatenate(
        [jnp.max(x_pool, axis=1), jnp.mean(x_pool, axis=1)], axis=1)

    RP1, RP2 = S * KK, N * S
    RP = RP1 + RP2
    RP_pad = _ceil(RP + 1, NSC * 2048) * NSC * 2048
    spread_p = (jnp.arange(S * ES, dtype=jnp.int32) % RP).reshape(S, ES)
    pd_dst = jnp.where(emask, gids * KK + nd, spread_p).reshape(-1)
    cnt_dst = RP1 + (orig_idx.astype(jnp.int32) * S + gids).reshape(-1)
    pc_dst = jnp.concatenate([pd_dst, cnt_dst])
    pc_val = jnp.concatenate(
        [emask.astype(f32).reshape(-1), jnp.ones((S * NS,), f32)])
    pc_dst_pk, nch_p = _pack_flat(pc_dst, NT, ("spread", RP))
    pc_val_pk, _ = _pack_flat(pc_val, NT, 0.0)
    pc2 = _agg_scalars(pc_val_pk, pc_dst_pk, RP_pad, nch_p)
    pc = pc2[:RP] + pc2[RP_pad:RP_pad + RP]
    dinv_p = ((pc[:RP1] + 1.0) ** -0.5).reshape(S, KK)
    counts = pc[RP1:].reshape(N, S)

    dvp = dinv_p.reshape(-1, 1)
    SPRp = 8
    zrow_id = S * KK
    spread = (jnp.arange(S * ES, dtype=jnp.int32) % (SPRp * KK)).reshape(
        S, ES)
    srcp_flat = jnp.where(emask, gids * KK + ns, zrow_id).reshape(-1)
    dstp_flat = jnp.where(emask, (gids % SPRp) * KK + nd, spread).reshape(-1)
    srcp_pk, nch_pp = _pack_grouped(srcp_flat, 2, zrow_id)
    dstp_pk, _ = _pack_grouped(dstp_flat, 2, 0)
    zp = x_pool.reshape(S * KK, NH) * dvp
    zp_t = jnp.concatenate([zp, jnp.zeros((8, NH), f32)])
    yp = _agg_rows_grouped(zp_t, srcp_pk, dstp_pk, KK, SPRp, 1, NH, nch_pp)
    xs = jax.nn.relu(dvp * _mm(yp + zp, Ws, 2000) + bs)
    xs = xs.reshape(S, KK, NH)
    emb2 = jnp.concatenate(
        [jnp.max(xs, axis=1), jnp.mean(xs, axis=1)], axis=1)
    sub_embs = emb1 + emb2

    att = _mhsa(sub_embs, Wqkv, bqkv, Wout, bout, 2)[:, 0, :]

    NG_pad = _ceil(N + 1, 256) * 256
    g_src = jnp.arange(S * NS, dtype=jnp.int32)
    g_dst = orig_idx.astype(jnp.int32).reshape(-1)
    g_src_pk, nch_g = _pack_flat(g_src, NT, S * NS)
    g_dst_pk, _ = _pack_flat(g_dst, NT, ("spread", N))
    g2 = _agg_rows_full(
        jnp.concatenate([xn.reshape(S * NS, NH), zpad8]), g_src_pk,
        g_dst_pk, NG_pad, NH, nch_g)
    global0 = g2[:N] + g2[NG_pad:NG_pad + N]
    global1 = _mm(counts, att, 2000)
    global_emb = jnp.concatenate([global0, global1], axis=1)

    xwf = _mm(global_emb, Wf, 2000)
    zf = jnp.pad(xwf * dinv_f[:, None], ((0, 8), (0, 128 - C)))
    f_src_pk, nch_f = _pack_flat(edge_index[0], NT, N)
    f_dst_pk, _ = _pack_flat(edge_index[1], NT, ("spread", N))
    f2 = _agg_rows_full(zf, f_src_pk, f_dst_pk, NG_pad, 128, nch_f)
    aggf = f2[:N, :C] + f2[NG_pad:NG_pad + N, :C]
    logits = dinv_f[:, None] * (aggf + xwf * dinv_f[:, None]) + bf
    return jax.nn.log_softmax(logits, axis=-1), global_emb


# speedup vs baseline: 1.8958x; 1.8958x over previous
import functools

import jax
import jax.numpy as jnp
from jax import lax
from jax.experimental import pallas as pl
from jax.experimental.pallas import tpu as pltpu
from jax.experimental.pallas import tpu_sc as plsc

NC = 2
NSC = 16
NT = NC * NSC
CH = 128


def _ceil(a, b):
    return -(-a // b)


def _pad_block(n, pad, dtype):
    if isinstance(pad, tuple):
        return (jnp.arange(n, dtype=jnp.int32) % pad[1]).astype(dtype)
    return jnp.full((n,), pad, dtype)


def _pack_flat(arr, n_tiles, pad):
    e = arr.shape[0]
    per = _ceil(_ceil(e, n_tiles), CH) * CH
    total = n_tiles * per
    arr = jnp.concatenate([arr, _pad_block(total - e, pad, arr.dtype)])
    return arr.reshape(n_tiles, per // CH, CH), per // CH


def _pack_grouped(arr, n_groups, pad):
    eg = arr.shape[0] // n_groups
    per = _ceil(_ceil(eg, NSC), CH) * CH
    a = arr.reshape(n_groups, NSC, eg // NSC)
    padb = jnp.broadcast_to(
        _pad_block(per - eg // NSC, pad, arr.dtype),
        (n_groups, NSC, per - eg // NSC))
    a = jnp.concatenate([a, padb], axis=2)
    return a.reshape(n_groups, NSC, per // CH, CH), per // CH


def _sc_mesh():
    return plsc.VectorSubcoreMesh(core_axis_name="c", subcore_axis_name="s",
                                  num_cores=NC, num_subcores=NSC)


def _zero_fill_2d(zrow, acc, row0, n_blocks, f):
    def zbody(i, carry):
        for j in range(f // 16):
            zrow[i, pl.ds(16 * j, 16)] = jnp.zeros((16,), jnp.float32)
        return carry

    lax.fori_loop(0, 16, zbody, 0)

    def fbody(i, carry):
        pltpu.sync_copy(zrow, acc.at[pl.ds(row0 + 16 * i, 16)])
        return carry

    lax.fori_loop(0, n_blocks, fbody, 0)


def _agg_rows_full(table, src_pk, dst_pk, r_pad, f, n_chunks):
    rows_per = r_pad // NSC

    @functools.partial(
        pl.kernel,
        out_type=jax.ShapeDtypeStruct((2 * r_pad, f), jnp.float32),
        mesh=_sc_mesh(),
        scratch_types=[
            pltpu.VMEM((n_chunks, CH), jnp.int32),
            pltpu.VMEM((n_chunks, CH), jnp.int32),
            pltpu.VMEM((CH, f), jnp.float32),
            pltpu.VMEM((16, f), jnp.float32),
            pltpu.VMEM((16, f), jnp.float32),
            pltpu.VMEM_SHARED((r_pad, f), jnp.float32),
            pltpu.SemaphoreType.DMA,
        ],
    )
    def k(table_h, src_h, dst_h, out_h, sidx, didx, buf, zrow, tbuf, acc,
          sem):
        cid = lax.axis_index("c")
        sid = lax.axis_index("s")
        wid = cid * NSC + sid
        _zero_fill_2d(zrow, acc, sid * rows_per, rows_per // 16, f)
        pltpu.sync_copy(src_h.at[wid], sidx)
        pltpu.sync_copy(dst_h.at[wid], didx)
        plsc.subcore_barrier()

        def body(c, carry):
            pltpu.async_copy(table_h.at[sidx.at[c]], buf, sem).wait()
            pltpu.sync_copy(buf, acc.at[didx.at[c]], add=True)
            return carry

        lax.fori_loop(0, n_chunks, body, 0)
        plsc.subcore_barrier()

        def obody(i, carry):
            pltpu.sync_copy(
                acc.at[pl.ds(sid * rows_per + 16 * i, 16)], tbuf)
            pltpu.sync_copy(
                tbuf,
                out_h.at[pl.ds(cid * r_pad + sid * rows_per + 16 * i, 16)])
            return carry

        lax.fori_loop(0, rows_per // 16, obody, 0)

    return k(table, src_pk, dst_pk)


def _agg_rows_grouped(table, src_pk, dst_pk, ns_out, spr, nr, f, n_chunks):
    r_acc = spr * ns_out
    r_pad = _ceil(r_acc + 1, 256) * 256
    rows_per = r_pad // NSC
    out_per = r_acc // NSC
    src_pk = src_pk.reshape(2 * nr * NSC, n_chunks, CH)
    dst_pk = dst_pk.reshape(2 * nr * NSC, n_chunks, CH)

    @functools.partial(
        pl.kernel,
        out_type=jax.ShapeDtypeStruct((2 * nr * r_pad, f), jnp.float32),
        mesh=_sc_mesh(),
        scratch_types=[
            pltpu.VMEM((n_chunks, CH), jnp.int32),
            pltpu.VMEM((n_chunks, CH), jnp.int32),
            pltpu.VMEM((CH, f), jnp.float32),
            pltpu.VMEM((16, f), jnp.float32),
            pltpu.VMEM((16, f), jnp.float32),
            pltpu.VMEM_SHARED((r_pad, f), jnp.float32),
            pltpu.SemaphoreType.DMA,
        ],
    )
    def k(table_h, src_h, dst_h, out_h, sidx, didx, buf, zrow, tbuf, acc,
          sem):
        cid = lax.axis_index("c")
        sid = lax.axis_index("s")
        for r in range(nr):
            grp = cid * nr + r
            _zero_fill_2d(zrow, acc, sid * rows_per, rows_per // 16, f)
            pltpu.sync_copy(src_h.at[grp * NSC + sid], sidx)
            pltpu.sync_copy(dst_h.at[grp * NSC + sid], didx)
            plsc.subcore_barrier()

            def body(c, carry):
                pltpu.async_copy(table_h.at[sidx.at[c]], buf, sem).wait()
                pltpu.sync_copy(buf, acc.at[didx.at[c]], add=True)
                return carry

            lax.fori_loop(0, n_chunks, body, 0)
            plsc.subcore_barrier()

            def obody(i, carry):
                pltpu.sync_copy(
                    acc.at[pl.ds(sid * rows_per + 16 * i, 16)], tbuf)
                pltpu.sync_copy(
                    tbuf,
                    out_h.at[pl.ds(grp * r_pad + sid * rows_per + 16 * i,
                                   16)])
                return carry

            lax.fori_loop(0, rows_per // 16, obody, 0)
            plsc.subcore_barrier()

    out = k(table, src_pk, dst_pk)
    return out.reshape(2 * nr, r_pad, f)[:, :r_acc, :].reshape(
        2 * nr * r_acc, f)


def _agg_scalars(vals_pk, dst_pk, r_pad, n_chunks):
    rows_per = r_pad // NSC
    assert rows_per % 2048 == 0

    @functools.partial(
        pl.kernel,
        out_type=jax.ShapeDtypeStruct((2 * r_pad,), jnp.float32),
        mesh=_sc_mesh(),
        scratch_types=[
            pltpu.VMEM((n_chunks, CH), jnp.float32),
            pltpu.VMEM((n_chunks, CH), jnp.int32),
            pltpu.VMEM((2048,), jnp.float32),
            pltpu.VMEM((2048,), jnp.float32),
            pltpu.VMEM_SHARED((r_pad,), jnp.float32),
            pltpu.SemaphoreType.DMA,
        ],
    )
    def k(vals_h, dst_h, out_h, vbuf, didx, zbuf, obuf, acc, sem):
        cid = lax.axis_index("c")
        sid = lax.axis_index("s")
        wid = cid * NSC + sid

        def zbody(i, carry):
            zbuf[pl.ds(16 * i, 16)] = jnp.zeros((16,), jnp.float32)
            return carry

        lax.fori_loop(0, 128, zbody, 0)

        def fbody(i, carry):
            pltpu.sync_copy(
                zbuf, acc.at[pl.ds(sid * rows_per + 2048 * i, 2048)])
            return carry

        lax.fori_loop(0, rows_per // 2048, fbody, 0)
        pltpu.sync_copy(vals_h.at[wid], vbuf)
        pltpu.sync_copy(dst_h.at[wid], didx)
        plsc.subcore_barrier()

        def body(c, carry):
            pltpu.sync_copy(vbuf.at[c], acc.at[didx.at[c]], add=True)
            return carry

        lax.fori_loop(0, n_chunks, body, 0)
        plsc.subcore_barrier()

        def obody(i, carry):
            pltpu.sync_copy(
                acc.at[pl.ds(sid * rows_per + 2048 * i, 2048)], obuf)
            pltpu.sync_copy(
                obuf,
                out_h.at[pl.ds(cid * r_pad + sid * rows_per + 2048 * i,
                               2048)])
            return carry

        lax.fori_loop(0, rows_per // 2048, obody, 0)

    return k(vals_pk, dst_pk)


def _mm_kernel(x_ref, w_ref, o_ref):
    o_ref[...] = jnp.dot(x_ref[...], w_ref[...],
                         preferred_element_type=jnp.float32)


def _mm(x, w, block_m):
    m, kdim = x.shape
    _, n = w.shape
    assert m % block_m == 0
    return pl.pallas_call(
        _mm_kernel,
        out_shape=jax.ShapeDtypeStruct((m, n), jnp.float32),
        grid=(m // block_m,),
        in_specs=[
            pl.BlockSpec((block_m, kdim), lambda i: (i, 0)),
            pl.BlockSpec((kdim, n), lambda i: (0, 0)),
        ],
        out_specs=pl.BlockSpec((block_m, n), lambda i: (i, 0)),
    )(x, w)


def _mhsa(x, Wqkv, bqkv, Wout, bout, H):
    S, E = x.shape
    hd = E // H
    qkv = (x @ Wqkv.T + bqkv).reshape(S, 1, 3, H, hd).transpose(2, 0, 3, 1, 4)
    q, k, v = qkv[0], qkv[1], qkv[2]
    attn = jax.nn.softmax((q @ jnp.swapaxes(k, -2, -1)) / (hd ** 0.5), axis=-1)
    out = (attn @ v).transpose(0, 2, 1, 3).reshape(S, 1, E)
    return out @ Wout.T + bout


def kernel(x, edge_index, sub_x, sub_edge_index, orig_idx, W1, b1, W2, b2,
           Wp, bp, Ws, bs, Wqkv, bqkv, Wout, bout, Wf, bf):
    N = x.shape[0]
    S, NS, D = sub_x.shape
    ES = sub_edge_index.shape[2]
    NH = W1.shape[1]
    C = Wf.shape[1]
    KK = NS // 2
    f32 = jnp.float32

    src_s = sub_edge_index[:, 0, :]
    dst_s = sub_edge_index[:, 1, :]
    gids = jnp.arange(S, dtype=jnp.int32)[:, None]

    R1, R2 = S * NS, N
    RD = R1 + R2
    RD_pad = _ceil(RD + 1, NSC * 2048) * NSC * 2048
    dummy_d = RD
    dsub_flat = (gids * NS + dst_s).reshape(-1)
    deg_dst = jnp.concatenate([dsub_flat, R1 + edge_index[1]])
    deg_dst_pk, nch_d = _pack_flat(deg_dst, NT, ("spread", RD))
    ones_pk, _ = _pack_flat(jnp.ones(deg_dst.shape, f32), NT, 0.0)
    deg2 = _agg_scalars(ones_pk, deg_dst_pk, RD_pad, nch_d)
    deg_all = deg2[:RD] + deg2[RD_pad:RD_pad + RD] + 1.0
    dinv_sub = (deg_all[:R1] ** -0.5).reshape(S, NS)
    dinv_f = deg_all[R1:] ** -0.5

    dv = dinv_sub.reshape(-1, 1)
    SPR = 4
    NR = S // (2 * SPR)
    zrow_s = S * NS
    src_flat = (gids * NS + src_s).reshape(-1)
    dstm_flat = ((gids % SPR) * NS + dst_s).reshape(-1)
    src_pk, nch_s = _pack_grouped(src_flat, 2 * NR, zrow_s)
    dst_pk, _ = _pack_grouped(dstm_flat, 2 * NR, ("spread", SPR * NS))
    zpad8 = jnp.zeros((8, NH), f32)
    z0 = sub_x.reshape(S * NS, D) * dv
    y0 = _agg_rows_grouped(jnp.concatenate([z0, zpad8]), src_pk, dst_pk,
                           NS, SPR, NR, D, nch_s)
    x1 = jax.nn.relu(dv * _mm(y0 + z0, W1, 2000) + b1)

    Wcat = jnp.concatenate(
        [W2, Wp, jnp.zeros((NH, 7), f32)], axis=1)
    z1 = x1 * dv
    y1 = _agg_rows_grouped(jnp.concatenate([z1, zpad8]), src_pk, dst_pk,
                           NS, SPR, NR, NH, nch_s)
    h2 = dv * _mm(y1 + z1, Wcat, 2000)
    xn = jax.nn.relu(h2[:, :NH] + b2).reshape(S, NS, NH)
    score = (h2[:, NH] + bp[0]).reshape(S, NS)

    topv, perm = jax.lax.top_k(score, KK)
    x1r = x1.reshape(S, NS, NH)
    x_pool = jnp.take_along_axis(x1r, perm[:, :, None], axis=1) \
        * jnp.tanh(topv)[:, :, None]
    new_idx = jnp.full((S, NS), -1, jnp.int32).at[
        jnp.arange(S)[:, None], perm].set(
            jnp.broadcast_to(jnp.arange(KK, dtype=jnp.int32), (S, KK)))
    ns = jnp.take_along_axis(new_idx, src_s, axis=1)
    nd = jnp.take_along_axis(new_idx, dst_s, axis=1)
    emask = (ns >= 0) & (nd >= 0)
    ns = jnp.where(emask, ns, 0)
    nd = jnp.where(emask, nd, 0)

    emb1 = jnp.concatenate(
        [jnp.max(x_pool, axis=1), jnp.mean(x_pool, axis=1)], axis=1)

    RP1, RP2 = S * KK, N * S
    RP = RP1 + RP2
    RP_pad = _ceil(RP + 1, NSC * 2048) * NSC * 2048
    spread_p = (jnp.arange(S * ES, dtype=jnp.int32) % RP).reshape(S, ES)
    pd_dst = jnp.where(emask, gids * KK + nd, spread_p).reshape(-1)
    cnt_dst = RP1 + (orig_idx.astype(jnp.int32) * S + gids).reshape(-1)
    pc_dst = jnp.concatenate([pd_dst, cnt_dst])
    pc_val = jnp.concatenate(
        [emask.astype(f32).reshape(-1), jnp.ones((S * NS,), f32)])
    pc_dst_pk, nch_p = _pack_flat(pc_dst, NT, ("spread", RP))
    pc_val_pk, _ = _pack_flat(pc_val, NT, 0.0)
    pc2 = _agg_scalars(pc_val_pk, pc_dst_pk, RP_pad, nch_p)
    pc = pc2[:RP] + pc2[RP_pad:RP_pad + RP]
    dinv_p = ((pc[:RP1] + 1.0) ** -0.5).reshape(S, KK)
    counts = pc[RP1:].reshape(N, S)

    dvp = dinv_p.reshape(-1, 1)
    SPRp = 8
    ZR = 2048
    eidx = jnp.arange(S * ES, dtype=jnp.int32).reshape(S, ES)
    spread = eidx % (SPRp * KK)
    zspread = S * KK + (eidx % ZR)
    srcp_flat = jnp.where(emask, gids * KK + ns, zspread).reshape(-1)
    dstp_flat = jnp.where(emask, (gids % SPRp) * KK + nd, spread).reshape(-1)
    srcp_pk, nch_pp = _pack_grouped(srcp_flat, 2, S * KK)
    dstp_pk, _ = _pack_grouped(dstp_flat, 2, 0)
    zp = x_pool.reshape(S * KK, NH) * dvp
    zp_t = jnp.concatenate([zp, jnp.zeros((ZR, NH), f32)])
    yp = _agg_rows_grouped(zp_t, srcp_pk, dstp_pk, KK, SPRp, 1, NH, nch_pp)
    xs = jax.nn.relu(dvp * _mm(yp + zp, Ws, 2000) + bs)
    xs = xs.reshape(S, KK, NH)
    emb2 = jnp.concatenate(
        [jnp.max(xs, axis=1), jnp.mean(xs, axis=1)], axis=1)
    sub_embs = emb1 + emb2

    att = _mhsa(sub_embs, Wqkv, bqkv, Wout, bout, 2)[:, 0, :]

    NG_pad = _ceil(N + 1, 256) * 256
    g_src = jnp.arange(S * NS, dtype=jnp.int32)
    g_dst = orig_idx.astype(jnp.int32).reshape(-1)
    g_src_pk, nch_g = _pack_flat(g_src, NT, S * NS)
    g_dst_pk, _ = _pack_flat(g_dst, NT, ("spread", N))
    g2 = _agg_rows_full(
        jnp.concatenate([xn.reshape(S * NS, NH), zpad8]), g_src_pk,
        g_dst_pk, NG_pad, NH, nch_g)
    global0 = g2[:N] + g2[NG_pad:NG_pad + N]
    global1 = _mm(counts, att, 2000)
    global_emb = jnp.concatenate([global0, global1], axis=1)

    xwf = _mm(global_emb, Wf, 2000)
    zf = jnp.pad(xwf * dinv_f[:, None], ((0, 8), (0, 128 - C)))
    f_src_pk, nch_f = _pack_flat(edge_index[0], NT, N)
    f_dst_pk, _ = _pack_flat(edge_index[1], NT, ("spread", N))
    f2 = _agg_rows_full(zf, f_src_pk, f_dst_pk, NG_pad, 128, nch_f)
    aggf = f2[:N, :C] + f2[NG_pad:NG_pad + N, :C]
    logits = dinv_f[:, None] * (aggf + xwf * dinv_f[:, None]) + bf
    return jax.nn.log_softmax(logits, axis=-1), global_emb


# speedup vs baseline: 7.9121x; 4.1735x over previous
import functools

import jax
import jax.numpy as jnp
from jax import lax
from jax.experimental import pallas as pl
from jax.experimental.pallas import tpu as pltpu
from jax.experimental.pallas import tpu_sc as plsc

NC = 2
NSC = 16
NT = NC * NSC
CH = 128


def _ceil(a, b):
    return -(-a // b)


def _pad_block(n, pad, dtype):
    if isinstance(pad, tuple):
        return (jnp.arange(n, dtype=jnp.int32) % pad[1]).astype(dtype)
    return jnp.full((n,), pad, dtype)


def _pack_flat(arr, n_tiles, pad):
    e = arr.shape[0]
    per = _ceil(_ceil(e, n_tiles), CH) * CH
    total = n_tiles * per
    arr = jnp.concatenate([arr, _pad_block(total - e, pad, arr.dtype)])
    return arr.reshape(n_tiles, per // CH, CH), per // CH


def _pack_grouped(arr, n_groups, pad):
    eg = arr.shape[0] // n_groups
    per = _ceil(_ceil(eg, NSC), CH) * CH
    a = arr.reshape(n_groups, NSC, eg // NSC)
    padb = jnp.broadcast_to(
        _pad_block(per - eg // NSC, pad, arr.dtype),
        (n_groups, NSC, per - eg // NSC))
    a = jnp.concatenate([a, padb], axis=2)
    return a.reshape(n_groups, NSC, per // CH, CH), per // CH


def _sc_mesh():
    return plsc.VectorSubcoreMesh(core_axis_name="c", subcore_axis_name="s",
                                  num_cores=NC, num_subcores=NSC)


def _zero_fill_2d(zrow, acc, row0, n_blocks, f):
    def zbody(i, carry):
        for j in range(f // 16):
            zrow[i, pl.ds(16 * j, 16)] = jnp.zeros((16,), jnp.float32)
        return carry

    lax.fori_loop(0, 16, zbody, 0)

    def fbody(i, carry):
        pltpu.sync_copy(zrow, acc.at[pl.ds(row0 + 16 * i, 16)])
        return carry

    lax.fori_loop(0, n_blocks, fbody, 0)


def _agg_rows_full(table, src_pk, dst_pk, r_pad, f, n_chunks):
    rows_per = r_pad // NSC

    @functools.partial(
        pl.kernel,
        out_type=jax.ShapeDtypeStruct((2 * r_pad, f), jnp.float32),
        mesh=_sc_mesh(),
        scratch_types=[
            pltpu.VMEM((n_chunks, CH), jnp.int32),
            pltpu.VMEM((n_chunks, CH), jnp.int32),
            pltpu.VMEM((CH, f), jnp.float32),
            pltpu.VMEM((16, f), jnp.float32),
            pltpu.VMEM((16, f), jnp.float32),
            pltpu.VMEM_SHARED((r_pad, f), jnp.float32),
            pltpu.SemaphoreType.DMA,
        ],
    )
    def k(table_h, src_h, dst_h, out_h, sidx, didx, buf, zrow, tbuf, acc,
          sem):
        cid = lax.axis_index("c")
        sid = lax.axis_index("s")
        wid = cid * NSC + sid
        _zero_fill_2d(zrow, acc, sid * rows_per, rows_per // 16, f)
        pltpu.sync_copy(src_h.at[wid], sidx)
        pltpu.sync_copy(dst_h.at[wid], didx)
        plsc.subcore_barrier()

        def body(c, carry):
            pltpu.async_copy(table_h.at[sidx.at[c]], buf, sem).wait()
            pltpu.sync_copy(buf, acc.at[didx.at[c]], add=True)
            return carry

        lax.fori_loop(0, n_chunks, body, 0)
        plsc.subcore_barrier()

        def obody(i, carry):
            pltpu.sync_copy(
                acc.at[pl.ds(sid * rows_per + 16 * i, 16)], tbuf)
            pltpu.sync_copy(
                tbuf,
                out_h.at[pl.ds(cid * r_pad + sid * rows_per + 16 * i, 16)])
            return carry

        lax.fori_loop(0, rows_per // 16, obody, 0)

    return k(table, src_pk, dst_pk)


def _agg_rows_grouped(table, src_pk, dst_pk, ns_out, spr, nr, f, n_chunks):
    r_acc = spr * ns_out
    r_pad = _ceil(r_acc + 1, 256) * 256
    rows_per = r_pad // NSC
    out_per = r_acc // NSC
    src_pk = src_pk.reshape(2 * nr * NSC, n_chunks, CH)
    dst_pk = dst_pk.reshape(2 * nr * NSC, n_chunks, CH)

    @functools.partial(
        pl.kernel,
        out_type=jax.ShapeDtypeStruct((2 * nr * r_pad, f), jnp.float32),
        mesh=_sc_mesh(),
        scratch_types=[
            pltpu.VMEM((n_chunks, CH), jnp.int32),
            pltpu.VMEM((n_chunks, CH), jnp.int32),
            pltpu.VMEM((CH, f), jnp.float32),
            pltpu.VMEM((16, f), jnp.float32),
            pltpu.VMEM((16, f), jnp.float32),
            pltpu.VMEM_SHARED((r_pad, f), jnp.float32),
            pltpu.SemaphoreType.DMA,
        ],
    )
    def k(table_h, src_h, dst_h, out_h, sidx, didx, buf, zrow, tbuf, acc,
          sem):
        cid = lax.axis_index("c")
        sid = lax.axis_index("s")
        for r in range(nr):
            grp = cid * nr + r
            _zero_fill_2d(zrow, acc, sid * rows_per, rows_per // 16, f)
            pltpu.sync_copy(src_h.at[grp * NSC + sid], sidx)
            pltpu.sync_copy(dst_h.at[grp * NSC + sid], didx)
            plsc.subcore_barrier()

            def body(c, carry):
                pltpu.async_copy(table_h.at[sidx.at[c]], buf, sem).wait()
                pltpu.sync_copy(buf, acc.at[didx.at[c]], add=True)
                return carry

            lax.fori_loop(0, n_chunks, body, 0)
            plsc.subcore_barrier()

            def obody(i, carry):
                pltpu.sync_copy(
                    acc.at[pl.ds(sid * rows_per + 16 * i, 16)], tbuf)
                pltpu.sync_copy(
                    tbuf,
                    out_h.at[pl.ds(grp * r_pad + sid * rows_per + 16 * i,
                                   16)])
                return carry

            lax.fori_loop(0, rows_per // 16, obody, 0)
            plsc.subcore_barrier()

    out = k(table, src_pk, dst_pk)
    return out.reshape(2 * nr, r_pad, f)[:, :r_acc, :].reshape(
        2 * nr * r_acc, f)


def _agg_scalars(vals_pk, dst_pk, r_pad, n_chunks):
    rows_per = r_pad // NSC
    assert rows_per % 2048 == 0

    @functools.partial(
        pl.kernel,
        out_type=jax.ShapeDtypeStruct((2 * r_pad,), jnp.float32),
        mesh=_sc_mesh(),
        scratch_types=[
            pltpu.VMEM((n_chunks, CH), jnp.float32),
            pltpu.VMEM((n_chunks, CH), jnp.int32),
            pltpu.VMEM((2048,), jnp.float32),
            pltpu.VMEM((2048,), jnp.float32),
            pltpu.VMEM_SHARED((r_pad,), jnp.float32),
            pltpu.SemaphoreType.DMA,
        ],
    )
    def k(vals_h, dst_h, out_h, vbuf, didx, zbuf, obuf, acc, sem):
        cid = lax.axis_index("c")
        sid = lax.axis_index("s")
        wid = cid * NSC + sid

        def zbody(i, carry):
            zbuf[pl.ds(16 * i, 16)] = jnp.zeros((16,), jnp.float32)
            return carry

        lax.fori_loop(0, 128, zbody, 0)

        def fbody(i, carry):
            pltpu.sync_copy(
                zbuf, acc.at[pl.ds(sid * rows_per + 2048 * i, 2048)])
            return carry

        lax.fori_loop(0, rows_per // 2048, fbody, 0)
        pltpu.sync_copy(vals_h.at[wid], vbuf)
        pltpu.sync_copy(dst_h.at[wid], didx)
        plsc.subcore_barrier()

        def body(c, carry):
            pltpu.sync_copy(vbuf.at[c], acc.at[didx.at[c]], add=True)
            return carry

        lax.fori_loop(0, n_chunks, body, 0)
        plsc.subcore_barrier()

        def obody(i, carry):
            pltpu.sync_copy(
                acc.at[pl.ds(sid * rows_per + 2048 * i, 2048)], obuf)
            pltpu.sync_copy(
                obuf,
                out_h.at[pl.ds(cid * r_pad + sid * rows_per + 2048 * i,
                               2048)])
            return carry

        lax.fori_loop(0, rows_per // 2048, obody, 0)

    return k(vals_pk, dst_pk)


def _remap_edges(new_idx_f, src_f, dst_f, S, NS, ES, KK, ZR, RP):
    EH = ES * S // NT
    SPRp = 8
    SKK = S * KK

    @functools.partial(
        pl.kernel,
        out_type=(
            jax.ShapeDtypeStruct((S * ES,), jnp.int32),
            jax.ShapeDtypeStruct((S * ES,), jnp.int32),
            jax.ShapeDtypeStruct((S * ES,), jnp.int32),
            jax.ShapeDtypeStruct((S * ES,), jnp.float32),
        ),
        mesh=_sc_mesh(),
        scratch_types=[
            pltpu.VMEM((NS,), jnp.int32),
            pltpu.VMEM((EH,), jnp.int32),
            pltpu.VMEM((EH,), jnp.int32),
            pltpu.VMEM((EH,), jnp.int32),
            pltpu.VMEM((EH,), jnp.int32),
            pltpu.VMEM((EH,), jnp.int32),
            pltpu.VMEM((EH,), jnp.float32),
        ],
        compiler_params=pltpu.CompilerParams(needs_layout_passes=False),
    )
    def k(tab_h, src_h, dst_h, o1_h, o2_h, o3_h, o4_h,
          table, sbuf, dbuf, o1, o2, o3, o4):
        cid = lax.axis_index("c")
        sid = lax.axis_index("s")
        wid = cid * NSC + sid
        g = wid // 2
        gbase = wid * EH
        pltpu.sync_copy(tab_h.at[pl.ds(g * NS, NS)], table)
        pltpu.sync_copy(src_h.at[pl.ds(gbase, EH)], sbuf)
        pltpu.sync_copy(dst_h.at[pl.ds(gbase, EH)], dbuf)

        def body(i, carry):
            sl = pl.ds(16 * i, 16)
            nsv = plsc.load_gather(table, [sbuf[sl]])
            ndv = plsc.load_gather(table, [dbuf[sl]])
            m = (nsv >= 0) & (ndv >= 0)
            e = gbase + 16 * i + lax.broadcasted_iota(jnp.int32, (16,), 0)
            o1[sl] = jnp.where(m, g * KK + nsv, SKK + e % ZR)
            o2[sl] = jnp.where(m, (g % SPRp) * KK + ndv, e % (SPRp * KK))
            o3[sl] = jnp.where(m, g * KK + ndv, e % RP)
            o4[sl] = m.astype(jnp.float32)
            return carry

        lax.fori_loop(0, EH // 16, body, 0)
        pltpu.sync_copy(o1, o1_h.at[pl.ds(gbase, EH)])
        pltpu.sync_copy(o2, o2_h.at[pl.ds(gbase, EH)])
        pltpu.sync_copy(o3, o3_h.at[pl.ds(gbase, EH)])
        pltpu.sync_copy(o4, o4_h.at[pl.ds(gbase, EH)])

    return k(new_idx_f, src_f, dst_f)


def _mm_kernel(x_ref, w_ref, o_ref):
    o_ref[...] = jnp.dot(x_ref[...], w_ref[...],
                         preferred_element_type=jnp.float32)


def _mm(x, w, block_m):
    m, kdim = x.shape
    _, n = w.shape
    assert m % block_m == 0
    return pl.pallas_call(
        _mm_kernel,
        out_shape=jax.ShapeDtypeStruct((m, n), jnp.float32),
        grid=(m // block_m,),
        in_specs=[
            pl.BlockSpec((block_m, kdim), lambda i: (i, 0)),
            pl.BlockSpec((kdim, n), lambda i: (0, 0)),
        ],
        out_specs=pl.BlockSpec((block_m, n), lambda i: (i, 0)),
    )(x, w)


def _mhsa(x, Wqkv, bqkv, Wout, bout, H):
    S, E = x.shape
    hd = E // H
    qkv = (x @ Wqkv.T + bqkv).reshape(S, 1, 3, H, hd).transpose(2, 0, 3, 1, 4)
    q, k, v = qkv[0], qkv[1], qkv[2]
    attn = jax.nn.softmax((q @ jnp.swapaxes(k, -2, -1)) / (hd ** 0.5), axis=-1)
    out = (attn @ v).transpose(0, 2, 1, 3).reshape(S, 1, E)
    return out @ Wout.T + bout


def kernel(x, edge_index, sub_x, sub_edge_index, orig_idx, W1, b1, W2, b2,
           Wp, bp, Ws, bs, Wqkv, bqkv, Wout, bout, Wf, bf):
    N = x.shape[0]
    S, NS, D = sub_x.shape
    ES = sub_edge_index.shape[2]
    NH = W1.shape[1]
    C = Wf.shape[1]
    KK = NS // 2
    f32 = jnp.float32

    src_s = sub_edge_index[:, 0, :]
    dst_s = sub_edge_index[:, 1, :]
    gids = jnp.arange(S, dtype=jnp.int32)[:, None]

    R1, R2 = S * NS, N
    RD = R1 + R2
    RD_pad = _ceil(RD + 1, NSC * 2048) * NSC * 2048
    dummy_d = RD
    dsub_flat = (gids * NS + dst_s).reshape(-1)
    deg_dst = jnp.concatenate([dsub_flat, R1 + edge_index[1]])
    deg_dst_pk, nch_d = _pack_flat(deg_dst, NT, ("spread", RD))
    ones_pk, _ = _pack_flat(jnp.ones(deg_dst.shape, f32), NT, 0.0)
    deg2 = _agg_scalars(ones_pk, deg_dst_pk, RD_pad, nch_d)
    deg_all = deg2[:RD] + deg2[RD_pad:RD_pad + RD] + 1.0
    dinv_sub = (deg_all[:R1] ** -0.5).reshape(S, NS)
    dinv_f = deg_all[R1:] ** -0.5

    dv = dinv_sub.reshape(-1, 1)
    SPR = 4
    NR = S // (2 * SPR)
    zrow_s = S * NS
    src_flat = (gids * NS + src_s).reshape(-1)
    dstm_flat = ((gids % SPR) * NS + dst_s).reshape(-1)
    src_pk, nch_s = _pack_grouped(src_flat, 2 * NR, zrow_s)
    dst_pk, _ = _pack_grouped(dstm_flat, 2 * NR, ("spread", SPR * NS))
    zpad8 = jnp.zeros((8, NH), f32)
    z0 = sub_x.reshape(S * NS, D) * dv
    y0 = _agg_rows_grouped(jnp.concatenate([z0, zpad8]), src_pk, dst_pk,
                           NS, SPR, NR, D, nch_s)
    x1 = jax.nn.relu(dv * _mm(y0 + z0, W1, 2000) + b1)

    Wcat = jnp.concatenate(
        [W2, Wp, jnp.zeros((NH, 7), f32)], axis=1)
    z1 = x1 * dv
    y1 = _agg_rows_grouped(jnp.concatenate([z1, zpad8]), src_pk, dst_pk,
                           NS, SPR, NR, NH, nch_s)
    h2 = dv * _mm(y1 + z1, Wcat, 2000)
    xn = jax.nn.relu(h2[:, :NH] + b2).reshape(S, NS, NH)
    score = (h2[:, NH] + bp[0]).reshape(S, NS)

    topv, perm = jax.lax.top_k(score, KK)
    x1r = x1.reshape(S, NS, NH)
    x_pool = jnp.take_along_axis(x1r, perm[:, :, None], axis=1) \
        * jnp.tanh(topv)[:, :, None]
    new_idx = jnp.full((S, NS), -1, jnp.int32).at[
        jnp.arange(S)[:, None], perm].set(
            jnp.broadcast_to(jnp.arange(KK, dtype=jnp.int32), (S, KK)))

    emb1 = jnp.concatenate(
        [jnp.max(x_pool, axis=1), jnp.mean(x_pool, axis=1)], axis=1)

    RP1, RP2 = S * KK, N * S
    RP = RP1 + RP2
    RP_pad = _ceil(RP + 1, NSC * 2048) * NSC * 2048
    ZR = 2048
    srcp_flat, dstp_flat, pd_dst, pd_val = _remap_edges(
        new_idx.reshape(-1), src_s.reshape(-1), dst_s.reshape(-1),
        S, NS, ES, KK, ZR, RP)

    cnt_dst = RP1 + (orig_idx.astype(jnp.int32) * S + gids).reshape(-1)
    pc_dst = jnp.concatenate([pd_dst, cnt_dst])
    pc_val = jnp.concatenate([pd_val, jnp.ones((S * NS,), f32)])
    pc_dst_pk, nch_p = _pack_flat(pc_dst, NT, ("spread", RP))
    pc_val_pk, _ = _pack_flat(pc_val, NT, 0.0)
    pc2 = _agg_scalars(pc_val_pk, pc_dst_pk, RP_pad, nch_p)
    pc = pc2[:RP] + pc2[RP_pad:RP_pad + RP]
    dinv_p = ((pc[:RP1] + 1.0) ** -0.5).reshape(S, KK)
    counts = pc[RP1:].reshape(N, S)

    dvp = dinv_p.reshape(-1, 1)
    SPRp = 8
    srcp_pk, nch_pp = _pack_grouped(srcp_flat, 2, S * KK)
    dstp_pk, _ = _pack_grouped(dstp_flat, 2, 0)
    zp = x_pool.reshape(S * KK, NH) * dvp
    zp_t = jnp.concatenate([zp, jnp.zeros((ZR, NH), f32)])
    yp = _agg_rows_grouped(zp_t, srcp_pk, dstp_pk, KK, SPRp, 1, NH, nch_pp)
    xs = jax.nn.relu(dvp * _mm(yp + zp, Ws, 2000) + bs)
    xs = xs.reshape(S, KK, NH)
    emb2 = jnp.concatenate(
        [jnp.max(xs, axis=1), jnp.mean(xs, axis=1)], axis=1)
    sub_embs = emb1 + emb2

    att = _mhsa(sub_embs, Wqkv, bqkv, Wout, bout, 2)[:, 0, :]

    NG_pad = _ceil(N + 1, 256) * 256
    g_src = jnp.arange(S * NS, dtype=jnp.int32)
    g_dst = orig_idx.astype(jnp.int32).reshape(-1)
    g_src_pk, nch_g = _pack_flat(g_src, NT, S * NS)
    g_dst_pk, _ = _pack_flat(g_dst, NT, ("spread", N))
    g2 = _agg_rows_full(
        jnp.concatenate([xn.reshape(S * NS, NH), zpad8]), g_src_pk,
        g_dst_pk, NG_pad, NH, nch_g)
    global0 = g2[:N] + g2[NG_pad:NG_pad + N]
    global1 = _mm(counts, att, 2000)
    global_emb = jnp.concatenate([global0, global1], axis=1)

    xwf = _mm(global_emb, Wf, 2000)
    zf = jnp.pad(xwf * dinv_f[:, None], ((0, 8), (0, 128 - C)))
    f_src_pk, nch_f = _pack_flat(edge_index[0], NT, N)
    f_dst_pk, _ = _pack_flat(edge_index[1], NT, ("spread", N))
    f2 = _agg_rows_full(zf, f_src_pk, f_dst_pk, NG_pad, 128, nch_f)
    aggf = f2[:N, :C] + f2[NG_pad:NG_pad + N, :C]
    logits = dinv_f[:, None] * (aggf + xwf * dinv_f[:, None]) + bf
    return jax.nn.log_softmax(logits, axis=-1), global_emb
